# baseline (speedup 1.0000x reference)
import jax
import jax.numpy as jnp
from jax.experimental import pallas as pl
from jax.experimental.pallas import tpu as pltpu

C_N = 2
K = 10


def _kern(
    ids_hbm,
    emb_hbm,
    pe_ref,
    ue_hbm,
    mask_hbm,
    ids_vmem,
    emb_buf,
    ue_buf,
    mask_vmem,
    pex_ref,
    ids_sem,
    in_sem,
    out_sem,
    mout_sem,
):
    N, D, B = emb_hbm.shape
    NC = N // C_N
    scale = float(D) ** 0.5

    ids_cp = pltpu.make_async_copy(ids_hbm, ids_vmem, ids_sem)
    ids_cp.start()

    def start_in(j, slot):
        pltpu.make_async_copy(
            emb_hbm.at[pl.ds(j * C_N, C_N)], emb_buf.at[slot], in_sem.at[slot]
        ).start()

    for s in range(K):
        start_in(s, s)

    pex_ref[...] = jnp.transpose(pe_ref[...], (1, 0))[:, :, None]
    ids_cp.wait()

    def body(r, carry):
        for slot in range(K):
            j = r * K + slot
            n0 = j * C_N
            pltpu.make_async_copy(
                emb_hbm.at[pl.ds(n0, C_N)], emb_buf.at[slot], in_sem.at[slot]
            ).wait()

            @pl.when(j >= K)
            def _():
                p0 = (j - K) * C_N
                pltpu.make_async_copy(
                    ue_buf.at[slot], ue_hbm.at[pl.ds(p0, C_N)], out_sem.at[slot]
                ).wait()

            m = (ids_vmem[j] != 0).astype(jnp.float32)
            mask_vmem[j] = m
            pe3 = pex_ref[pl.ds(n0, C_N)]
            ue_buf[slot] = (emb_buf[slot] * scale + pe3) * m[:, None, :]

            pltpu.make_async_copy(
                ue_buf.at[slot], ue_hbm.at[pl.ds(n0, C_N)], out_sem.at[slot]
            ).start()

            @pl.when(j + K < NC)
            def _():
                start_in(j + K, slot)

        return carry

    jax.lax.fori_loop(0, NC // K, body, 0)

    mask_cp = pltpu.make_async_copy(mask_vmem, mask_hbm, mout_sem)
    mask_cp.start()

    for s in range(K):
        j = NC - K + s
        slot = j % K
        n0 = j * C_N
        pltpu.make_async_copy(
            ue_buf.at[slot], ue_hbm.at[pl.ds(n0, C_N)], out_sem.at[slot]
        ).wait()
    mask_cp.wait()


def kernel(past_lengths, past_ids, past_embeddings, past_payloads, pos_emb):
    B, N = past_ids.shape
    D = past_embeddings.shape[-1]
    idsT = past_ids.T.reshape(N // C_N, C_N, B)
    embT = jnp.transpose(past_embeddings, (1, 2, 0))
    peT = pos_emb.T
    ueT, maskT = pl.pallas_call(
        _kern,
        in_specs=[
            pl.BlockSpec(memory_space=pltpu.HBM),
            pl.BlockSpec(memory_space=pltpu.HBM),
            pl.BlockSpec(memory_space=pltpu.VMEM),
        ],
        out_specs=[
            pl.BlockSpec(memory_space=pltpu.HBM),
            pl.BlockSpec(memory_space=pltpu.HBM),
        ],
        out_shape=[
            jax.ShapeDtypeStruct((N, D, B), jnp.float32),
            jax.ShapeDtypeStruct((N // C_N, C_N, B), jnp.float32),
        ],
        scratch_shapes=[
            pltpu.VMEM((N // C_N, C_N, B), jnp.int32),
            pltpu.VMEM((K, C_N, D, B), jnp.float32),
            pltpu.VMEM((K, C_N, D, B), jnp.float32),
            pltpu.VMEM((N // C_N, C_N, B), jnp.float32),
            pltpu.VMEM((N, D, 1), jnp.float32),
            pltpu.SemaphoreType.DMA,
            pltpu.SemaphoreType.DMA((K,)),
            pltpu.SemaphoreType.DMA((K,)),
            pltpu.SemaphoreType.DMA,
        ],
        compiler_params=pltpu.CompilerParams(
            vmem_limit_bytes=100 * 1024 * 1024,
        ),
    )(idsT, embT, peT)
    ue = jnp.transpose(ueT, (2, 0, 1))
    mask = maskT.reshape(N, B).T[..., None]
    return (past_lengths, ue, mask)


# speedup vs baseline: 1.0520x; 1.0520x over previous
import jax
import jax.numpy as jnp
from jax.experimental import pallas as pl
from jax.experimental.pallas import tpu as pltpu

C_N = 4
K = 5


def _kern(
    ids_hbm,
    emb_hbm,
    pe_ref,
    ue_hbm,
    mask_hbm,
    ids_buf,
    emb_buf,
    ue_buf,
    mask_buf,
    pex_ref,
    ids_sem,
    in_sem,
    out_sem,
    mout_sem,
):
    N, D, B = emb_hbm.shape
    NC = N // C_N
    scale = float(D) ** 0.5

    def start_in(j, slot):
        pltpu.make_async_copy(
            emb_hbm.at[pl.ds(j * C_N, C_N)], emb_buf.at[slot], in_sem.at[slot]
        ).start()
        pltpu.make_async_copy(
            ids_hbm.at[pl.ds(j * C_N, C_N)], ids_buf.at[slot], ids_sem.at[slot]
        ).start()

    for s in range(K):
        start_in(s, s)

    pex_ref[...] = jnp.transpose(pe_ref[...], (1, 0))[:, :, None]

    def body(r, carry):
        for slot in range(K):
            j = r * K + slot
            n0 = j * C_N
            pltpu.make_async_copy(
                emb_hbm.at[pl.ds(n0, C_N)], emb_buf.at[slot], in_sem.at[slot]
            ).wait()
            pltpu.make_async_copy(
                ids_hbm.at[pl.ds(n0, C_N)], ids_buf.at[slot], ids_sem.at[slot]
            ).wait()

            @pl.when(j >= K)
            def _():
                p0 = (j - K) * C_N
                pltpu.make_async_copy(
                    ue_buf.at[slot], ue_hbm.at[pl.ds(p0, C_N)], out_sem.at[slot]
                ).wait()
                pltpu.make_async_copy(
                    mask_buf.at[slot],
                    mask_hbm.at[pl.ds(p0, C_N)],
                    mout_sem.at[slot],
                ).wait()

            m = (ids_buf[slot] != 0).astype(jnp.float32)
            mask_buf[slot] = m
            pe3 = pex_ref[pl.ds(n0, C_N)]
            ue_buf[slot] = (emb_buf[slot] * scale + pe3) * m[:, None, :]

            pltpu.make_async_copy(
                ue_buf.at[slot], ue_hbm.at[pl.ds(n0, C_N)], out_sem.at[slot]
            ).start()
            pltpu.make_async_copy(
                mask_buf.at[slot], mask_hbm.at[pl.ds(n0, C_N)], mout_sem.at[slot]
            ).start()

            @pl.when(j + K < NC)
            def _():
                start_in(j + K, slot)

        return carry

    jax.lax.fori_loop(0, NC // K, body, 0)

    for s in range(K):
        j = NC - K + s
        slot = j % K
        n0 = j * C_N
        pltpu.make_async_copy(
            ue_buf.at[slot], ue_hbm.at[pl.ds(n0, C_N)], out_sem.at[slot]
        ).wait()
        pltpu.make_async_copy(
            mask_buf.at[slot], mask_hbm.at[pl.ds(n0, C_N)], mout_sem.at[slot]
        ).wait()


def kernel(past_lengths, past_ids, past_embeddings, past_payloads, pos_emb):
    B, N = past_ids.shape
    D = past_embeddings.shape[-1]
    idsT = past_ids.T
    embT = jnp.transpose(past_embeddings, (1, 2, 0))
    peT = pos_emb.T
    ueT, maskT = pl.pallas_call(
        _kern,
        in_specs=[
            pl.BlockSpec(memory_space=pltpu.HBM),
            pl.BlockSpec(memory_space=pltpu.HBM),
            pl.BlockSpec(memory_space=pltpu.VMEM),
        ],
        out_specs=[
            pl.BlockSpec(memory_space=pltpu.HBM),
            pl.BlockSpec(memory_space=pltpu.HBM),
        ],
        out_shape=[
            jax.ShapeDtypeStruct((N, D, B), jnp.float32),
            jax.ShapeDtypeStruct((N, B), jnp.float32),
        ],
        scratch_shapes=[
            pltpu.VMEM((K, C_N, B), jnp.int32),
            pltpu.VMEM((K, C_N, D, B), jnp.float32),
            pltpu.VMEM((K, C_N, D, B), jnp.float32),
            pltpu.VMEM((K, C_N, B), jnp.float32),
            pltpu.VMEM((N, D, 1), jnp.float32),
            pltpu.SemaphoreType.DMA((K,)),
            pltpu.SemaphoreType.DMA((K,)),
            pltpu.SemaphoreType.DMA((K,)),
            pltpu.SemaphoreType.DMA((K,)),
        ],
        compiler_params=pltpu.CompilerParams(
            vmem_limit_bytes=100 * 1024 * 1024,
        ),
    )(idsT, embT, peT)
    ue = jnp.transpose(ueT, (2, 0, 1))
    mask = maskT.T[..., None]
    return (past_lengths, ue, mask)
